# baseline (speedup 1.0000x reference)
import functools

import jax
import jax.numpy as jnp
from jax import lax
from jax.experimental import pallas as pl
from jax.experimental.pallas import tpu as pltpu
from jax.experimental.pallas import tpu_sc as plsc

N = 10000
E = 640000
NCLS = 16

BN = 128
NP = 10240
NT = NP + 64
NW = 32
CH = 128
KCH = 160
EP = NW * KCH * CH


def _cnn_block(fe_ref, fo_ref, bcat_ref, brow_ref, w1_ref, b1_ref, w2_ref,
               b2_ref, out_ref):
    Fe = fe_ref[...]
    Fo = fo_ref[...]
    rows = BN * 10
    zpl = jnp.zeros((1, BN, 64), jnp.float32)
    fo_dn = jnp.concatenate([zpl, Fo[:-1]], axis=0)
    fe_up = jnp.concatenate([Fe[1:], zpl], axis=0)
    ge = jnp.concatenate([fo_dn, Fe, Fo],
                         axis=2).reshape(rows, 192).astype(jnp.bfloat16)
    go = jnp.concatenate([Fe, Fo, fe_up],
                         axis=2).reshape(rows, 192).astype(jnp.bfloat16)
    ye = jnp.dot(ge, bcat_ref[...], preferred_element_type=jnp.float32)
    yo = jnp.dot(go, bcat_ref[...], preferred_element_type=jnp.float32)
    yh = jnp.maximum(jnp.maximum(ye, yo) + brow_ref[...], 0.0)
    yw = jnp.maximum(yh, pltpu.roll(yh, shift=2047, axis=1))
    y3 = yw.reshape(10, BN, 2048)
    acc = jnp.broadcast_to(b1_ref[...], (BN, 32))
    for k in range(10):
        acc = acc + jnp.dot(y3[k], w1_ref[k],
                            preferred_element_type=jnp.float32)
    z = jnp.maximum(acc, 0.0)
    x = jnp.dot(z, w2_ref[...], preferred_element_type=jnp.float32)
    out_ref[...] = jnp.maximum(x + b2_ref[...], 0.0)


def _cnn(feats_e, feats_o, bcat, brow, w1p, b1r, w2p, b2r):
    return pl.pallas_call(
        _cnn_block,
        grid=(NP // BN,),
        in_specs=[
            pl.BlockSpec((10, BN, 64), lambda i: (0, i, 0)),
            pl.BlockSpec((10, BN, 64), lambda i: (0, i, 0)),
            pl.BlockSpec((192, 2048), lambda i: (0, 0)),
            pl.BlockSpec((1, 2048), lambda i: (0, 0)),
            pl.BlockSpec((10, 2048, 32), lambda i: (0, 0, 0)),
            pl.BlockSpec((1, 32), lambda i: (0, 0)),
            pl.BlockSpec((32, 32), lambda i: (0, 0)),
            pl.BlockSpec((1, 32), lambda i: (0, 0)),
        ],
        out_specs=pl.BlockSpec((BN, 32), lambda i: (i, 0)),
        out_shape=jax.ShapeDtypeStruct((NP, 32), jnp.float32),
    )(feats_e, feats_o, bcat, brow, w1p, b1r, w2p, b2r)


def _make_scatter(D, with_deg):
    mesh = plsc.VectorSubcoreMesh(core_axis_name="c", subcore_axis_name="s")
    out_type = [jax.ShapeDtypeStruct((2, NT, D), jnp.float32)]
    NB = 8
    HB = NB // 2
    scratch = [
        pltpu.VMEM((KCH, CH), jnp.int32),
        pltpu.VMEM((KCH, CH), jnp.int32),
    ]
    scratch += [pltpu.VMEM((CH, D), jnp.float32) for _ in range(NB)]
    scratch += [pltpu.VMEM_SHARED((NT, D), jnp.float32)]
    scratch += [pltpu.SemaphoreType.DMA for _ in range(2 * NB)]
    if with_deg:
        out_type.append(jax.ShapeDtypeStruct((2, NT, 8), jnp.float32))
        scratch += [
            pltpu.VMEM((CH, 8), jnp.float32),
            pltpu.VMEM_SHARED((NT, 8), jnp.float32),
        ]
        scratch += [pltpu.SemaphoreType.DMA for _ in range(NB)]

    def body(*refs):
        if with_deg:
            (x_hbm, src_hbm, dst_hbm, z_hbm, z8_hbm, ones_hbm,
             agg_out, deg_out, src_v, dst_v) = refs[:10]
            msg = list(refs[10:10 + NB])
            agg_sh = refs[10 + NB]
            gsem = list(refs[11 + NB:11 + 2 * NB])
            ssem = list(refs[11 + 2 * NB:11 + 3 * NB])
            ones_v = refs[11 + 3 * NB]
            deg_sh = refs[12 + 3 * NB]
            dsem = list(refs[13 + 3 * NB:13 + 4 * NB])
        else:
            (x_hbm, src_hbm, dst_hbm, z_hbm,
             agg_out, src_v, dst_v) = refs[:7]
            msg = list(refs[7:7 + NB])
            agg_sh = refs[7 + NB]
            gsem = list(refs[8 + NB:8 + 2 * NB])
            ssem = list(refs[8 + 2 * NB:8 + 3 * NB])
        c = lax.axis_index("c")
        s = lax.axis_index("s")
        wid = s * 2 + c
        pltpu.sync_copy(src_hbm.at[wid], src_v)
        pltpu.sync_copy(dst_hbm.at[wid], dst_v)
        if with_deg:
            pltpu.sync_copy(ones_hbm, ones_v)

        @pl.when(s == 0)
        def _init():
            pltpu.sync_copy(z_hbm, agg_sh)
            if with_deg:
                pltpu.sync_copy(z8_hbm, deg_sh)

        plsc.subcore_barrier()

        for b0 in range(HB):
            pltpu.async_copy(x_hbm.at[src_v.at[b0]], msg[b0], gsem[b0])

        def do_chunk(t, b):
            b2 = (b + HB) % NB
            pltpu.make_async_copy(x_hbm.at[src_v.at[t]], msg[b],
                                  gsem[b]).wait()
            pltpu.async_copy(msg[b], agg_sh.at[dst_v.at[t]], ssem[b],
                             add=True)
            if with_deg:
                pltpu.async_copy(ones_v, deg_sh.at[dst_v.at[t]], dsem[b],
                                 add=True)

            @pl.when(t >= HB)
            def _wait_prev():
                pltpu.make_async_copy(msg[b2], agg_sh.at[dst_v.at[t]],
                                      ssem[b2]).wait()
                if with_deg:
                    pltpu.make_async_copy(ones_v, deg_sh.at[dst_v.at[t]],
                                          dsem[b2]).wait()

            @pl.when(t + HB < KCH)
            def _next_gather():
                pltpu.async_copy(x_hbm.at[src_v.at[t + HB]], msg[b2],
                                 gsem[b2])

        def step(tt, carry):
            for b in range(NB):
                do_chunk(tt * NB + b, b)
            return carry

        lax.fori_loop(0, KCH // NB, step, 0)
        for b in range(HB, NB):
            pltpu.make_async_copy(msg[b], agg_sh.at[dst_v.at[0]],
                                  ssem[b]).wait()
            if with_deg:
                pltpu.make_async_copy(ones_v, deg_sh.at[dst_v.at[0]],
                                      dsem[b]).wait()
        plsc.subcore_barrier()

        @pl.when(s == 0)
        def _flush():
            pltpu.sync_copy(agg_sh, agg_out.at[c])
            if with_deg:
                pltpu.sync_copy(deg_sh, deg_out.at[c])

    return pl.kernel(
        body,
        out_type=tuple(out_type) if with_deg else out_type[0],
        mesh=mesh,
        scratch_types=scratch,
        compiler_params=pltpu.CompilerParams(use_tc_tiling_on_sc=False),
    )


def _combine1_block(x_ref, a_ref, d_ref, s1s_ref, s1n_ref, s1b_ref,
                    s2n_ref, s2s_ref, s2b_ref, p_ref, self2_ref):
    agg = a_ref[0] + a_ref[1]
    deg = d_ref[0, :, 0:1] + d_ref[1, :, 0:1]
    rdeg = 1.0 / jnp.maximum(deg, 1.0)
    h = jnp.dot(x_ref[...], s1s_ref[...], preferred_element_type=jnp.float32)
    h = h + jnp.dot(agg * rdeg, s1n_ref[...],
                    preferred_element_type=jnp.float32)
    h = jnp.maximum(h + s1b_ref[...], 0.0)
    p_ref[...] = jnp.dot(h, s2n_ref[...], preferred_element_type=jnp.float32)
    self2_ref[...] = (jnp.dot(h, s2s_ref[...],
                              preferred_element_type=jnp.float32)
                      + s2b_ref[...])


def _combine1(x, aggp, degp, s1sT, s1nT, s1b, s2nT, s2sT, s2b):
    B = 1280
    return pl.pallas_call(
        _combine1_block,
        grid=(NP // B,),
        in_specs=[
            pl.BlockSpec((B, 32), lambda i: (i, 0)),
            pl.BlockSpec((2, B, 32), lambda i: (0, i, 0)),
            pl.BlockSpec((2, B, 8), lambda i: (0, i, 0)),
            pl.BlockSpec((32, 64), lambda i: (0, 0)),
            pl.BlockSpec((32, 64), lambda i: (0, 0)),
            pl.BlockSpec((1, 64), lambda i: (0, 0)),
            pl.BlockSpec((64, NCLS), lambda i: (0, 0)),
            pl.BlockSpec((64, NCLS), lambda i: (0, 0)),
            pl.BlockSpec((1, NCLS), lambda i: (0, 0)),
        ],
        out_specs=[
            pl.BlockSpec((B, NCLS), lambda i: (i, 0)),
            pl.BlockSpec((B, NCLS), lambda i: (i, 0)),
        ],
        out_shape=[
            jax.ShapeDtypeStruct((NP, NCLS), jnp.float32),
            jax.ShapeDtypeStruct((NP, NCLS), jnp.float32),
        ],
    )(x, aggp, degp, s1sT, s1nT, s1b, s2nT, s2sT, s2b)


def _combine2_block(self2_ref, a_ref, d_ref, out_ref):
    agg = a_ref[0] + a_ref[1]
    deg = d_ref[0, :, 0:1] + d_ref[1, :, 0:1]
    rdeg = 1.0 / jnp.maximum(deg, 1.0)
    out_ref[...] = self2_ref[...] + agg * rdeg


def _combine2(self2, aggp, degp):
    B = 1280
    return pl.pallas_call(
        _combine2_block,
        grid=(NP // B,),
        in_specs=[
            pl.BlockSpec((B, NCLS), lambda i: (i, 0)),
            pl.BlockSpec((2, B, NCLS), lambda i: (0, i, 0)),
            pl.BlockSpec((2, B, 8), lambda i: (0, i, 0)),
        ],
        out_specs=pl.BlockSpec((B, NCLS), lambda i: (i, 0)),
        out_shape=jax.ShapeDtypeStruct((NP, NCLS), jnp.float32),
    )(self2, aggp, degp)


def kernel(features, edge_index, conv_w, conv_b, lin1_w, lin1_b, lin2_w,
           lin2_b, s1_self, s1_neigh, s1_b, s2_self, s2_neigh, s2_b):
    f32 = jnp.float32

    wp = jnp.arange(64)[:, None]
    ww = jnp.arange(64)[None, :]
    off = wp - ww + 1
    valid = (off >= 0) & (off <= 2)
    offc = jnp.clip(off, 0, 2)
    bds = []
    for dh in range(3):
        tap = conv_w[:, 0, dh, :]
        M = tap[:, offc]
        M = jnp.where(valid[None], M, 0.0)
        bds.append(jnp.transpose(M, (1, 0, 2)).reshape(64, 2048))
    bcat = jnp.concatenate(bds, axis=0)
    brow = jnp.repeat(conv_b, 64).reshape(1, 2048)

    cols = jnp.arange(2048)
    obase = (cols // 64) * 320 + (cols % 64) // 2
    even = (cols % 64) % 2 == 0
    w1p = jnp.stack([
        jnp.where(even[:, None], lin1_w[:, obase + ph * 32].T, 0.0)
        for ph in range(10)
    ])
    b1r = lin1_b.reshape(1, 32)
    w2p = lin2_w.T
    b2r = lin2_b.reshape(1, 32)

    feats = jnp.pad(features, ((0, NP - N), (0, 0), (0, 0)))
    feats_e = feats[:, 0::2, :].transpose(1, 0, 2)
    feats_o = feats[:, 1::2, :].transpose(1, 0, 2)

    x = _cnn(feats_e, feats_o, bcat.astype(jnp.bfloat16), brow, w1p, b1r,
             w2p, b2r)

    pad = EP - E
    srcp = jnp.concatenate([edge_index[0],
                            jnp.zeros((pad,), jnp.int32)]).reshape(NW, KCH, CH)
    dstp = jnp.concatenate([edge_index[1],
                            jnp.full((pad,), NP, jnp.int32)]).reshape(NW, KCH, CH)

    z32 = jnp.zeros((NT, 32), f32)
    z16 = jnp.zeros((NT, 16), f32)
    z8 = jnp.zeros((NT, 8), f32)
    ones8 = jnp.ones((CH, 8), f32)

    agg1p, degp = _make_scatter(32, True)(x, srcp, dstp, z32, z8, ones8)

    p, self2 = _combine1(x, agg1p, degp, s1_self.T, s1_neigh.T,
                         s1_b.reshape(1, 64), s2_neigh.T, s2_self.T,
                         s2_b.reshape(1, NCLS))

    agg2p = _make_scatter(16, False)(p, srcp, dstp, z16)

    out = _combine2(self2, agg2p, degp)
    return out[:N]


# speedup vs baseline: 1.0521x; 1.0521x over previous
import functools

import jax
import jax.numpy as jnp
from jax import lax
from jax.experimental import pallas as pl
from jax.experimental.pallas import tpu as pltpu
from jax.experimental.pallas import tpu_sc as plsc

N = 10000
E = 640000
NCLS = 16

BN = 128
NP = 10240
NT = NP + 64
NW = 32
CH = 128
KCH = 160
EP = NW * KCH * CH


def _cnn_block(fe_ref, fo_ref, bcat_ref, brow_ref, w1_ref, b1_ref, w2_ref,
               b2_ref, out_ref):
    Fe = fe_ref[...]
    Fo = fo_ref[...]
    rows = BN * 10
    zpl = jnp.zeros((1, BN, 64), jnp.float32)
    fo_dn = jnp.concatenate([zpl, Fo[:-1]], axis=0)
    fe_up = jnp.concatenate([Fe[1:], zpl], axis=0)
    ge = jnp.concatenate([fo_dn, Fe, Fo],
                         axis=2).reshape(rows, 192).astype(jnp.bfloat16)
    go = jnp.concatenate([Fe, Fo, fe_up],
                         axis=2).reshape(rows, 192).astype(jnp.bfloat16)
    ye = jnp.dot(ge, bcat_ref[...], preferred_element_type=jnp.float32)
    yo = jnp.dot(go, bcat_ref[...], preferred_element_type=jnp.float32)
    yh = jnp.maximum(jnp.maximum(ye, yo) + brow_ref[...], 0.0)
    yw = jnp.maximum(yh, pltpu.roll(yh, shift=2047, axis=1))
    y3 = yw.reshape(10, BN, 2048)
    acc = jnp.broadcast_to(b1_ref[...], (BN, 32))
    for k in range(10):
        acc = acc + jnp.dot(y3[k], w1_ref[k],
                            preferred_element_type=jnp.float32)
    z = jnp.maximum(acc, 0.0)
    x = jnp.dot(z, w2_ref[...], preferred_element_type=jnp.float32)
    out_ref[...] = jnp.maximum(x + b2_ref[...], 0.0)


def _cnn(feats_e, feats_o, bcat, brow, w1p, b1r, w2p, b2r):
    return pl.pallas_call(
        _cnn_block,
        grid=(NP // BN,),
        in_specs=[
            pl.BlockSpec((10, BN, 64), lambda i: (0, i, 0)),
            pl.BlockSpec((10, BN, 64), lambda i: (0, i, 0)),
            pl.BlockSpec((192, 2048), lambda i: (0, 0)),
            pl.BlockSpec((1, 2048), lambda i: (0, 0)),
            pl.BlockSpec((10, 2048, 32), lambda i: (0, 0, 0)),
            pl.BlockSpec((1, 32), lambda i: (0, 0)),
            pl.BlockSpec((32, 32), lambda i: (0, 0)),
            pl.BlockSpec((1, 32), lambda i: (0, 0)),
        ],
        out_specs=pl.BlockSpec((BN, 32), lambda i: (i, 0)),
        out_shape=jax.ShapeDtypeStruct((NP, 32), jnp.float32),
    )(feats_e, feats_o, bcat, brow, w1p, b1r, w2p, b2r)


def _make_scatter(D):
    mesh = plsc.VectorSubcoreMesh(core_axis_name="c", subcore_axis_name="s")
    out_type = jax.ShapeDtypeStruct((2, NT, D), jnp.float32)
    NB = 8
    HB = NB // 2
    scratch = [
        pltpu.VMEM((KCH, CH), jnp.int32),
        pltpu.VMEM((KCH, CH), jnp.int32),
    ]
    scratch += [pltpu.VMEM((CH, D), jnp.float32) for _ in range(NB)]
    scratch += [pltpu.VMEM_SHARED((NT, D), jnp.float32)]
    scratch += [pltpu.SemaphoreType.DMA for _ in range(2 * NB)]

    def body(*refs):
        (x_hbm, src_hbm, dst_hbm, z_hbm,
         agg_out, src_v, dst_v) = refs[:7]
        msg = list(refs[7:7 + NB])
        agg_sh = refs[7 + NB]
        gsem = list(refs[8 + NB:8 + 2 * NB])
        ssem = list(refs[8 + 2 * NB:8 + 3 * NB])
        c = lax.axis_index("c")
        s = lax.axis_index("s")
        wid = s * 2 + c
        pltpu.sync_copy(src_hbm.at[wid], src_v)
        pltpu.sync_copy(dst_hbm.at[wid], dst_v)

        @pl.when(s == 0)
        def _init():
            pltpu.sync_copy(z_hbm, agg_sh)

        plsc.subcore_barrier()

        for b0 in range(HB):
            pltpu.async_copy(x_hbm.at[src_v.at[b0]], msg[b0], gsem[b0])

        def do_chunk(t, b):
            b2 = (b + HB) % NB
            pltpu.make_async_copy(x_hbm.at[src_v.at[t]], msg[b],
                                  gsem[b]).wait()
            pltpu.async_copy(msg[b], agg_sh.at[dst_v.at[t]], ssem[b],
                             add=True)

            @pl.when(t >= HB)
            def _wait_prev():
                pltpu.make_async_copy(msg[b2], agg_sh.at[dst_v.at[t]],
                                      ssem[b2]).wait()

            @pl.when(t + HB < KCH)
            def _next_gather():
                pltpu.async_copy(x_hbm.at[src_v.at[t + HB]], msg[b2],
                                 gsem[b2])

        def step(tt, carry):
            for b in range(NB):
                do_chunk(tt * NB + b, b)
            return carry

        lax.fori_loop(0, KCH // NB, step, 0)
        for b in range(HB, NB):
            pltpu.make_async_copy(msg[b], agg_sh.at[dst_v.at[0]],
                                  ssem[b]).wait()
        plsc.subcore_barrier()

        @pl.when(s == 0)
        def _flush():
            pltpu.sync_copy(agg_sh, agg_out.at[c])

    return pl.kernel(
        body,
        out_type=out_type,
        mesh=mesh,
        scratch_types=scratch,
        compiler_params=pltpu.CompilerParams(use_tc_tiling_on_sc=False),
    )


def _make_deg():
    mesh = plsc.VectorSubcoreMesh(core_axis_name="c", subcore_axis_name="s")
    NB = 4
    scratch = [
        pltpu.VMEM((KCH, CH), jnp.int32),
        pltpu.VMEM((CH, 8), jnp.float32),
        pltpu.VMEM_SHARED((NT, 8), jnp.float32),
    ]
    scratch += [pltpu.SemaphoreType.DMA for _ in range(NB)]

    def body(*refs):
        (dst_hbm, z8_hbm, ones_hbm, deg_out, dst_v, ones_v, deg_sh) = refs[:7]
        dsem = list(refs[7:7 + NB])
        c = lax.axis_index("c")
        s = lax.axis_index("s")
        wid = s * 2 + c
        pltpu.sync_copy(dst_hbm.at[wid], dst_v)
        pltpu.sync_copy(ones_hbm, ones_v)

        @pl.when(s == 0)
        def _init():
            pltpu.sync_copy(z8_hbm, deg_sh)

        plsc.subcore_barrier()

        def do_chunk(t, b):
            @pl.when(t >= NB)
            def _wait_prev():
                pltpu.make_async_copy(ones_v, deg_sh.at[dst_v.at[t]],
                                      dsem[b]).wait()

            pltpu.async_copy(ones_v, deg_sh.at[dst_v.at[t]], dsem[b],
                             add=True)

        def step(tt, carry):
            for b in range(NB):
                do_chunk(tt * NB + b, b)
            return carry

        lax.fori_loop(0, KCH // NB, step, 0)
        for b in range(NB):
            pltpu.make_async_copy(ones_v, deg_sh.at[dst_v.at[0]],
                                  dsem[b]).wait()
        plsc.subcore_barrier()

        @pl.when(s == 0)
        def _flush():
            pltpu.sync_copy(deg_sh, deg_out.at[c])

    return pl.kernel(
        body,
        out_type=jax.ShapeDtypeStruct((2, NT, 8), jnp.float32),
        mesh=mesh,
        scratch_types=scratch,
        compiler_params=pltpu.CompilerParams(use_tc_tiling_on_sc=False),
    )


def _combine1_block(x_ref, a_ref, d_ref, s1s_ref, s1n_ref, s1b_ref,
                    s2n_ref, s2s_ref, s2b_ref, p_ref, self2_ref):
    agg = a_ref[0] + a_ref[1]
    deg = d_ref[0, :, 0:1] + d_ref[1, :, 0:1]
    rdeg = 1.0 / jnp.maximum(deg, 1.0)
    h = jnp.dot(x_ref[...], s1s_ref[...], preferred_element_type=jnp.float32)
    h = h + jnp.dot(agg * rdeg, s1n_ref[...],
                    preferred_element_type=jnp.float32)
    h = jnp.maximum(h + s1b_ref[...], 0.0)
    p_ref[...] = jnp.dot(h, s2n_ref[...], preferred_element_type=jnp.float32)
    self2_ref[...] = (jnp.dot(h, s2s_ref[...],
                              preferred_element_type=jnp.float32)
                      + s2b_ref[...])


def _combine1(x, aggp, degp, s1sT, s1nT, s1b, s2nT, s2sT, s2b):
    B = 1280
    return pl.pallas_call(
        _combine1_block,
        grid=(NP // B,),
        in_specs=[
            pl.BlockSpec((B, 32), lambda i: (i, 0)),
            pl.BlockSpec((2, B, 32), lambda i: (0, i, 0)),
            pl.BlockSpec((2, B, 8), lambda i: (0, i, 0)),
            pl.BlockSpec((32, 64), lambda i: (0, 0)),
            pl.BlockSpec((32, 64), lambda i: (0, 0)),
            pl.BlockSpec((1, 64), lambda i: (0, 0)),
            pl.BlockSpec((64, NCLS), lambda i: (0, 0)),
            pl.BlockSpec((64, NCLS), lambda i: (0, 0)),
            pl.BlockSpec((1, NCLS), lambda i: (0, 0)),
        ],
        out_specs=[
            pl.BlockSpec((B, NCLS), lambda i: (i, 0)),
            pl.BlockSpec((B, NCLS), lambda i: (i, 0)),
        ],
        out_shape=[
            jax.ShapeDtypeStruct((NP, NCLS), jnp.float32),
            jax.ShapeDtypeStruct((NP, NCLS), jnp.float32),
        ],
    )(x, aggp, degp, s1sT, s1nT, s1b, s2nT, s2sT, s2b)


def _combine2_block(self2_ref, a_ref, d_ref, out_ref):
    agg = a_ref[0] + a_ref[1]
    deg = d_ref[0, :, 0:1] + d_ref[1, :, 0:1]
    rdeg = 1.0 / jnp.maximum(deg, 1.0)
    out_ref[...] = self2_ref[...] + agg * rdeg


def _combine2(self2, aggp, degp):
    B = 1280
    return pl.pallas_call(
        _combine2_block,
        grid=(NP // B,),
        in_specs=[
            pl.BlockSpec((B, NCLS), lambda i: (i, 0)),
            pl.BlockSpec((2, B, NCLS), lambda i: (0, i, 0)),
            pl.BlockSpec((2, B, 8), lambda i: (0, i, 0)),
        ],
        out_specs=pl.BlockSpec((B, NCLS), lambda i: (i, 0)),
        out_shape=jax.ShapeDtypeStruct((NP, NCLS), jnp.float32),
    )(self2, aggp, degp)


def kernel(features, edge_index, conv_w, conv_b, lin1_w, lin1_b, lin2_w,
           lin2_b, s1_self, s1_neigh, s1_b, s2_self, s2_neigh, s2_b):
    f32 = jnp.float32

    wp = jnp.arange(64)[:, None]
    ww = jnp.arange(64)[None, :]
    off = wp - ww + 1
    valid = (off >= 0) & (off <= 2)
    offc = jnp.clip(off, 0, 2)
    bds = []
    for dh in range(3):
        tap = conv_w[:, 0, dh, :]
        M = tap[:, offc]
        M = jnp.where(valid[None], M, 0.0)
        bds.append(jnp.transpose(M, (1, 0, 2)).reshape(64, 2048))
    bcat = jnp.concatenate(bds, axis=0)
    brow = jnp.repeat(conv_b, 64).reshape(1, 2048)

    cols = jnp.arange(2048)
    obase = (cols // 64) * 320 + (cols % 64) // 2
    even = (cols % 64) % 2 == 0
    w1p = jnp.stack([
        jnp.where(even[:, None], lin1_w[:, obase + ph * 32].T, 0.0)
        for ph in range(10)
    ])
    b1r = lin1_b.reshape(1, 32)
    w2p = lin2_w.T
    b2r = lin2_b.reshape(1, 32)

    feats = jnp.pad(features, ((0, NP - N), (0, 0), (0, 0)))
    feats_e = feats[:, 0::2, :].transpose(1, 0, 2)
    feats_o = feats[:, 1::2, :].transpose(1, 0, 2)

    x = _cnn(feats_e, feats_o, bcat.astype(jnp.bfloat16), brow, w1p, b1r,
             w2p, b2r)

    pad = EP - E
    srcp = jnp.concatenate([edge_index[0],
                            jnp.zeros((pad,), jnp.int32)]).reshape(NW, KCH, CH)
    dstp = jnp.concatenate([edge_index[1],
                            jnp.full((pad,), NP, jnp.int32)]).reshape(NW, KCH, CH)

    z32 = jnp.zeros((NT, 32), f32)
    z16 = jnp.zeros((NT, 16), f32)
    z8 = jnp.zeros((NT, 8), f32)
    ones8 = jnp.ones((CH, 8), f32)

    degp = _make_deg()(dstp, z8, ones8)
    agg1p = _make_scatter(32)(x, srcp, dstp, z32)

    p, self2 = _combine1(x, agg1p, degp, s1_self.T, s1_neigh.T,
                         s1_b.reshape(1, 64), s2_neigh.T, s2_self.T,
                         s2_b.reshape(1, NCLS))

    agg2p = _make_scatter(16)(p, srcp, dstp, z16)

    out = _combine2(self2, agg2p, degp)
    return out[:N]


# speedup vs baseline: 1.3520x; 1.2851x over previous
import functools

import jax
import jax.numpy as jnp
from jax import lax
from jax.experimental import pallas as pl
from jax.experimental.pallas import tpu as pltpu
from jax.experimental.pallas import tpu_sc as plsc

N = 10000
E = 640000
NCLS = 16

BN = 128
NP = 10240
NT = NP + 64
NW = 32
CH = 128
KCH = 160
EP = NW * KCH * CH


def _cnn_block(f_ref, bcat_ref, brow_ref, w1_ref, b1_ref, w2_ref,
               b2_ref, out_ref):
    Fe = f_ref[0]
    Fo = f_ref[1]
    rows = BN * 10
    zpl = jnp.zeros((1, BN, 64), jnp.float32)
    fo_dn = jnp.concatenate([zpl, Fo[:-1]], axis=0)
    fe_up = jnp.concatenate([Fe[1:], zpl], axis=0)
    ge = jnp.concatenate([fo_dn, Fe, Fo],
                         axis=2).reshape(rows, 192).astype(jnp.bfloat16)
    go = jnp.concatenate([Fe, Fo, fe_up],
                         axis=2).reshape(rows, 192).astype(jnp.bfloat16)
    ye = jnp.dot(ge, bcat_ref[...], preferred_element_type=jnp.float32)
    yo = jnp.dot(go, bcat_ref[...], preferred_element_type=jnp.float32)
    yh = jnp.maximum(jnp.maximum(ye, yo) + brow_ref[...], 0.0)
    yw = jnp.maximum(yh, pltpu.roll(yh, shift=2047, axis=1))
    y3 = yw.reshape(10, BN, 2048)
    acc = jnp.broadcast_to(b1_ref[...], (BN, 32))
    for k in range(10):
        acc = acc + jnp.dot(y3[k], w1_ref[k],
                            preferred_element_type=jnp.float32)
    z = jnp.maximum(acc, 0.0)
    x = jnp.dot(z, w2_ref[...], preferred_element_type=jnp.float32)
    out_ref[...] = jnp.maximum(x + b2_ref[...], 0.0)


def _cnn(feats_t, bcat, brow, w1p, b1r, w2p, b2r):
    return pl.pallas_call(
        _cnn_block,
        grid=(NP // BN,),
        in_specs=[
            pl.BlockSpec((2, 10, BN, 64), lambda i: (0, 0, i, 0)),
            pl.BlockSpec((192, 2048), lambda i: (0, 0)),
            pl.BlockSpec((1, 2048), lambda i: (0, 0)),
            pl.BlockSpec((10, 2048, 32), lambda i: (0, 0, 0)),
            pl.BlockSpec((1, 32), lambda i: (0, 0)),
            pl.BlockSpec((32, 32), lambda i: (0, 0)),
            pl.BlockSpec((1, 32), lambda i: (0, 0)),
        ],
        out_specs=pl.BlockSpec((BN, 32), lambda i: (i, 0)),
        out_shape=jax.ShapeDtypeStruct((NP, 32), jnp.float32),
    )(feats_t, bcat, brow, w1p, b1r, w2p, b2r)


def _make_scatter(D):
    mesh = plsc.VectorSubcoreMesh(core_axis_name="c", subcore_axis_name="s")
    out_type = jax.ShapeDtypeStruct((2, NT, D), jnp.float32)
    NB = 8
    HB = NB // 2
    scratch = [
        pltpu.VMEM((KCH, CH), jnp.int32),
        pltpu.VMEM((KCH, CH), jnp.int32),
    ]
    scratch += [pltpu.VMEM((CH, D), jnp.float32) for _ in range(NB)]
    scratch += [
        pltpu.VMEM_SHARED((NT, D), jnp.float32),
        pltpu.VMEM_SHARED((NP, D), jnp.float32),
    ]
    scratch += [pltpu.SemaphoreType.DMA for _ in range(2 * NB)]

    def body(*refs):
        (x_hbm, src_hbm, dst_hbm, z_hbm,
         agg_out, src_v, dst_v) = refs[:7]
        msg = list(refs[7:7 + NB])
        agg_sh = refs[7 + NB]
        x_sh = refs[8 + NB]
        gsem = list(refs[9 + NB:9 + 2 * NB])
        ssem = list(refs[9 + 2 * NB:9 + 3 * NB])
        c = lax.axis_index("c")
        s = lax.axis_index("s")
        wid = s * 2 + c
        pltpu.sync_copy(src_hbm.at[wid], src_v)
        pltpu.sync_copy(dst_hbm.at[wid], dst_v)
        rs = NP // 16
        pltpu.sync_copy(x_hbm.at[pl.ds(s * rs, rs)],
                        x_sh.at[pl.ds(s * rs, rs)])

        @pl.when(s == 0)
        def _init():
            pltpu.sync_copy(z_hbm, agg_sh)

        plsc.subcore_barrier()

        for b0 in range(HB):
            pltpu.async_copy(x_sh.at[src_v.at[b0]], msg[b0], gsem[b0])

        def do_chunk(t, b):
            b2 = (b + HB) % NB
            pltpu.make_async_copy(x_sh.at[src_v.at[t]], msg[b],
                                  gsem[b]).wait()
            pltpu.async_copy(msg[b], agg_sh.at[dst_v.at[t]], ssem[b],
                             add=True)

            @pl.when(t >= HB)
            def _wait_prev():
                pltpu.make_async_copy(msg[b2], agg_sh.at[dst_v.at[t]],
                                      ssem[b2]).wait()

            @pl.when(t + HB < KCH)
            def _next_gather():
                pltpu.async_copy(x_sh.at[src_v.at[t + HB]], msg[b2],
                                 gsem[b2])

        def step(tt, carry):
            for b in range(NB):
                do_chunk(tt * NB + b, b)
            return carry

        lax.fori_loop(0, KCH // NB, step, 0)
        for b in range(HB, NB):
            pltpu.make_async_copy(msg[b], agg_sh.at[dst_v.at[0]],
                                  ssem[b]).wait()
        plsc.subcore_barrier()

        @pl.when(s == 0)
        def _flush():
            pltpu.sync_copy(agg_sh, agg_out.at[c])

    return pl.kernel(
        body,
        out_type=out_type,
        mesh=mesh,
        scratch_types=scratch,
        compiler_params=pltpu.CompilerParams(use_tc_tiling_on_sc=False),
    )


def _make_deg():
    mesh = plsc.VectorSubcoreMesh(core_axis_name="c", subcore_axis_name="s")
    NB = 4
    scratch = [
        pltpu.VMEM((KCH, CH), jnp.int32),
        pltpu.VMEM((CH, 8), jnp.float32),
        pltpu.VMEM_SHARED((NT, 8), jnp.float32),
    ]
    scratch += [pltpu.SemaphoreType.DMA for _ in range(NB)]

    def body(*refs):
        (dst_hbm, z8_hbm, ones_hbm, deg_out, dst_v, ones_v, deg_sh) = refs[:7]
        dsem = list(refs[7:7 + NB])
        c = lax.axis_index("c")
        s = lax.axis_index("s")
        wid = s * 2 + c
        pltpu.sync_copy(dst_hbm.at[wid], dst_v)
        pltpu.sync_copy(ones_hbm, ones_v)

        @pl.when(s == 0)
        def _init():
            pltpu.sync_copy(z8_hbm, deg_sh)

        plsc.subcore_barrier()

        def do_chunk(t, b):
            @pl.when(t >= NB)
            def _wait_prev():
                pltpu.make_async_copy(ones_v, deg_sh.at[dst_v.at[t]],
                                      dsem[b]).wait()

            pltpu.async_copy(ones_v, deg_sh.at[dst_v.at[t]], dsem[b],
                             add=True)

        def step(tt, carry):
            for b in range(NB):
                do_chunk(tt * NB + b, b)
            return carry

        lax.fori_loop(0, KCH // NB, step, 0)
        for b in range(NB):
            pltpu.make_async_copy(ones_v, deg_sh.at[dst_v.at[0]],
                                  dsem[b]).wait()
        plsc.subcore_barrier()

        @pl.when(s == 0)
        def _flush():
            pltpu.sync_copy(deg_sh, deg_out.at[c])

    return pl.kernel(
        body,
        out_type=jax.ShapeDtypeStruct((2, NT, 8), jnp.float32),
        mesh=mesh,
        scratch_types=scratch,
        compiler_params=pltpu.CompilerParams(use_tc_tiling_on_sc=False),
    )


def _combine1_block(x_ref, a_ref, d_ref, s1s_ref, s1n_ref, s1b_ref,
                    s2n_ref, s2s_ref, s2b_ref, p_ref, self2_ref):
    agg = a_ref[0] + a_ref[1]
    deg = d_ref[0, :, 0:1] + d_ref[1, :, 0:1]
    rdeg = 1.0 / jnp.maximum(deg, 1.0)
    h = jnp.dot(x_ref[...], s1s_ref[...], preferred_element_type=jnp.float32)
    h = h + jnp.dot(agg * rdeg, s1n_ref[...],
                    preferred_element_type=jnp.float32)
    h = jnp.maximum(h + s1b_ref[...], 0.0)
    p_ref[...] = jnp.dot(h, s2n_ref[...], preferred_element_type=jnp.float32)
    self2_ref[...] = (jnp.dot(h, s2s_ref[...],
                              preferred_element_type=jnp.float32)
                      + s2b_ref[...])


def _combine1(x, aggp, degp, s1sT, s1nT, s1b, s2nT, s2sT, s2b):
    B = 1280
    return pl.pallas_call(
        _combine1_block,
        grid=(NP // B,),
        in_specs=[
            pl.BlockSpec((B, 32), lambda i: (i, 0)),
            pl.BlockSpec((2, B, 32), lambda i: (0, i, 0)),
            pl.BlockSpec((2, B, 8), lambda i: (0, i, 0)),
            pl.BlockSpec((32, 64), lambda i: (0, 0)),
            pl.BlockSpec((32, 64), lambda i: (0, 0)),
            pl.BlockSpec((1, 64), lambda i: (0, 0)),
            pl.BlockSpec((64, NCLS), lambda i: (0, 0)),
            pl.BlockSpec((64, NCLS), lambda i: (0, 0)),
            pl.BlockSpec((1, NCLS), lambda i: (0, 0)),
        ],
        out_specs=[
            pl.BlockSpec((B, NCLS), lambda i: (i, 0)),
            pl.BlockSpec((B, NCLS), lambda i: (i, 0)),
        ],
        out_shape=[
            jax.ShapeDtypeStruct((NP, NCLS), jnp.float32),
            jax.ShapeDtypeStruct((NP, NCLS), jnp.float32),
        ],
    )(x, aggp, degp, s1sT, s1nT, s1b, s2nT, s2sT, s2b)


def _combine2_block(self2_ref, a_ref, d_ref, out_ref):
    agg = a_ref[0] + a_ref[1]
    deg = d_ref[0, :, 0:1] + d_ref[1, :, 0:1]
    rdeg = 1.0 / jnp.maximum(deg, 1.0)
    out_ref[...] = self2_ref[...] + agg * rdeg


def _combine2(self2, aggp, degp):
    B = 1280
    return pl.pallas_call(
        _combine2_block,
        grid=(NP // B,),
        in_specs=[
            pl.BlockSpec((B, NCLS), lambda i: (i, 0)),
            pl.BlockSpec((2, B, NCLS), lambda i: (0, i, 0)),
            pl.BlockSpec((2, B, 8), lambda i: (0, i, 0)),
        ],
        out_specs=pl.BlockSpec((B, NCLS), lambda i: (i, 0)),
        out_shape=jax.ShapeDtypeStruct((NP, NCLS), jnp.float32),
    )(self2, aggp, degp)


def kernel(features, edge_index, conv_w, conv_b, lin1_w, lin1_b, lin2_w,
           lin2_b, s1_self, s1_neigh, s1_b, s2_self, s2_neigh, s2_b):
    f32 = jnp.float32

    wp = jnp.arange(64)[:, None]
    ww = jnp.arange(64)[None, :]
    off = wp - ww + 1
    valid = (off >= 0) & (off <= 2)
    offc = jnp.clip(off, 0, 2)
    bds = []
    for dh in range(3):
        tap = conv_w[:, 0, dh, :]
        M = tap[:, offc]
        M = jnp.where(valid[None], M, 0.0)
        bds.append(jnp.transpose(M, (1, 0, 2)).reshape(64, 2048))
    bcat = jnp.concatenate(bds, axis=0)
    brow = jnp.repeat(conv_b, 64).reshape(1, 2048)

    cols = jnp.arange(2048)
    obase = (cols // 64) * 320 + (cols % 64) // 2
    even = (cols % 64) % 2 == 0
    w1p = jnp.stack([
        jnp.where(even[:, None], lin1_w[:, obase + ph * 32].T, 0.0)
        for ph in range(10)
    ])
    b1r = lin1_b.reshape(1, 32)
    w2p = lin2_w.T
    b2r = lin2_b.reshape(1, 32)

    feats = jnp.pad(features, ((0, NP - N), (0, 0), (0, 0)))
    feats_t = feats.reshape(NP, 10, 2, 64).transpose(2, 1, 0, 3)

    x = _cnn(feats_t, bcat.astype(jnp.bfloat16), brow, w1p, b1r,
             w2p, b2r)

    pad = EP - E
    srcp = jnp.concatenate([edge_index[0],
                            jnp.zeros((pad,), jnp.int32)]).reshape(NW, KCH, CH)
    dstp = jnp.concatenate([edge_index[1],
                            jnp.full((pad,), NP, jnp.int32)]).reshape(NW, KCH, CH)

    z32 = jnp.zeros((NT, 32), f32)
    z16 = jnp.zeros((NT, 16), f32)
    z8 = jnp.zeros((NT, 8), f32)
    ones8 = jnp.ones((CH, 8), f32)

    degp = _make_deg()(dstp, z8, ones8)
    agg1p = _make_scatter(32)(x, srcp, dstp, z32)

    p, self2 = _combine1(x, agg1p, degp, s1_self.T, s1_neigh.T,
                         s1_b.reshape(1, 64), s2_neigh.T, s2_self.T,
                         s2_b.reshape(1, NCLS))

    agg2p = _make_scatter(16)(p, srcp, dstp, z16)

    out = _combine2(self2, agg2p, degp)
    return out[:N]


# speedup vs baseline: 1.4073x; 1.0409x over previous
import functools

import jax
import jax.numpy as jnp
from jax import lax
from jax.experimental import pallas as pl
from jax.experimental.pallas import tpu as pltpu
from jax.experimental.pallas import tpu_sc as plsc

N = 10000
E = 640000
NCLS = 16

BN = 128
NP = 10240
NT = NP + 64
NW = 32
CH = 128
KCH = 160
EP = NW * KCH * CH


def _cnn_block(f_ref, bcat_ref, brow_ref, w1_ref, b1_ref, w2_ref,
               b2_ref, out_ref):
    Fe = f_ref[0]
    Fo = f_ref[1]
    rows = BN * 10
    zpl = jnp.zeros((1, BN, 64), jnp.float32)
    fo_dn = jnp.concatenate([zpl, Fo[:-1]], axis=0)
    fe_up = jnp.concatenate([Fe[1:], zpl], axis=0)
    ge = jnp.concatenate([fo_dn, Fe, Fo],
                         axis=2).reshape(rows, 192).astype(jnp.bfloat16)
    go = jnp.concatenate([Fe, Fo, fe_up],
                         axis=2).reshape(rows, 192).astype(jnp.bfloat16)
    ye = jnp.dot(ge, bcat_ref[...], preferred_element_type=jnp.float32)
    yo = jnp.dot(go, bcat_ref[...], preferred_element_type=jnp.float32)
    yh = jnp.maximum(jnp.maximum(ye, yo) + brow_ref[...],
                     0.0).astype(jnp.bfloat16)
    yw = jnp.maximum(yh, pltpu.roll(yh, shift=2047, axis=1))
    y3 = yw.reshape(10, BN, 2048)
    acc = jnp.broadcast_to(b1_ref[...], (BN, 32))
    for k in range(10):
        acc = acc + jnp.dot(y3[k], w1_ref[k],
                            preferred_element_type=jnp.float32)
    z = jnp.maximum(acc, 0.0)
    x = jnp.dot(z, w2_ref[...], preferred_element_type=jnp.float32)
    out_ref[...] = jnp.maximum(x + b2_ref[...], 0.0)


def _cnn(feats_t, bcat, brow, w1p, b1r, w2p, b2r):
    return pl.pallas_call(
        _cnn_block,
        grid=(NP // BN,),
        in_specs=[
            pl.BlockSpec((2, 10, BN, 64), lambda i: (0, 0, i, 0)),
            pl.BlockSpec((192, 2048), lambda i: (0, 0)),
            pl.BlockSpec((1, 2048), lambda i: (0, 0)),
            pl.BlockSpec((10, 2048, 32), lambda i: (0, 0, 0)),
            pl.BlockSpec((1, 32), lambda i: (0, 0)),
            pl.BlockSpec((32, 32), lambda i: (0, 0)),
            pl.BlockSpec((1, 32), lambda i: (0, 0)),
        ],
        out_specs=pl.BlockSpec((BN, 32), lambda i: (i, 0)),
        out_shape=jax.ShapeDtypeStruct((NP, 32), jnp.float32),
    )(feats_t, bcat, brow, w1p, b1r, w2p, b2r)


def _make_scatter(D):
    mesh = plsc.VectorSubcoreMesh(core_axis_name="c", subcore_axis_name="s")
    out_type = jax.ShapeDtypeStruct((2, NT, D), jnp.float32)
    NB = 8
    HB = NB // 2
    scratch = [
        pltpu.VMEM((KCH, CH), jnp.int32),
        pltpu.VMEM((KCH, CH), jnp.int32),
    ]
    scratch += [pltpu.VMEM((CH, D), jnp.float32) for _ in range(NB)]
    scratch += [
        pltpu.VMEM_SHARED((NT, D), jnp.float32),
        pltpu.VMEM_SHARED((NP, D), jnp.float32),
    ]
    scratch += [pltpu.SemaphoreType.DMA for _ in range(2 * NB)]

    def body(*refs):
        (x_hbm, src_hbm, dst_hbm, z_hbm,
         agg_out, src_v, dst_v) = refs[:7]
        msg = list(refs[7:7 + NB])
        agg_sh = refs[7 + NB]
        x_sh = refs[8 + NB]
        gsem = list(refs[9 + NB:9 + 2 * NB])
        ssem = list(refs[9 + 2 * NB:9 + 3 * NB])
        c = lax.axis_index("c")
        s = lax.axis_index("s")
        wid = s * 2 + c
        pltpu.sync_copy(src_hbm.at[wid], src_v)
        pltpu.sync_copy(dst_hbm.at[wid], dst_v)
        rs = NP // 16
        pltpu.sync_copy(x_hbm.at[pl.ds(s * rs, rs)],
                        x_sh.at[pl.ds(s * rs, rs)])

        @pl.when(s == 0)
        def _init():
            pltpu.sync_copy(z_hbm, agg_sh)

        plsc.subcore_barrier()

        for b0 in range(HB):
            pltpu.async_copy(x_sh.at[src_v.at[b0]], msg[b0], gsem[b0])

        def do_chunk(t, b):
            b2 = (b + HB) % NB
            pltpu.make_async_copy(x_sh.at[src_v.at[t]], msg[b],
                                  gsem[b]).wait()
            pltpu.async_copy(msg[b], agg_sh.at[dst_v.at[t]], ssem[b],
                             add=True)

            @pl.when(t >= HB)
            def _wait_prev():
                pltpu.make_async_copy(msg[b2], agg_sh.at[dst_v.at[t]],
                                      ssem[b2]).wait()

            @pl.when(t + HB < KCH)
            def _next_gather():
                pltpu.async_copy(x_sh.at[src_v.at[t + HB]], msg[b2],
                                 gsem[b2])

        def step(tt, carry):
            for b in range(NB):
                do_chunk(tt * NB + b, b)
            return carry

        lax.fori_loop(0, KCH // NB, step, 0)
        for b in range(HB, NB):
            pltpu.make_async_copy(msg[b], agg_sh.at[dst_v.at[0]],
                                  ssem[b]).wait()
        plsc.subcore_barrier()

        @pl.when(s == 0)
        def _flush():
            pltpu.sync_copy(agg_sh, agg_out.at[c])

    return pl.kernel(
        body,
        out_type=out_type,
        mesh=mesh,
        scratch_types=scratch,
        compiler_params=pltpu.CompilerParams(use_tc_tiling_on_sc=False),
    )


def _make_deg():
    mesh = plsc.VectorSubcoreMesh(core_axis_name="c", subcore_axis_name="s")
    NB = 4
    scratch = [
        pltpu.VMEM((KCH, CH), jnp.int32),
        pltpu.VMEM((CH, 8), jnp.float32),
        pltpu.VMEM_SHARED((NT, 8), jnp.float32),
    ]
    scratch += [pltpu.SemaphoreType.DMA for _ in range(NB)]

    def body(*refs):
        (dst_hbm, z8_hbm, ones_hbm, deg_out, dst_v, ones_v, deg_sh) = refs[:7]
        dsem = list(refs[7:7 + NB])
        c = lax.axis_index("c")
        s = lax.axis_index("s")
        wid = s * 2 + c
        pltpu.sync_copy(dst_hbm.at[wid], dst_v)
        pltpu.sync_copy(ones_hbm, ones_v)

        @pl.when(s == 0)
        def _init():
            pltpu.sync_copy(z8_hbm, deg_sh)

        plsc.subcore_barrier()

        def do_chunk(t, b):
            @pl.when(t >= NB)
            def _wait_prev():
                pltpu.make_async_copy(ones_v, deg_sh.at[dst_v.at[t]],
                                      dsem[b]).wait()

            pltpu.async_copy(ones_v, deg_sh.at[dst_v.at[t]], dsem[b],
                             add=True)

        def step(tt, carry):
            for b in range(NB):
                do_chunk(tt * NB + b, b)
            return carry

        lax.fori_loop(0, KCH // NB, step, 0)
        for b in range(NB):
            pltpu.make_async_copy(ones_v, deg_sh.at[dst_v.at[0]],
                                  dsem[b]).wait()
        plsc.subcore_barrier()

        @pl.when(s == 0)
        def _flush():
            pltpu.sync_copy(deg_sh, deg_out.at[c])

    return pl.kernel(
        body,
        out_type=jax.ShapeDtypeStruct((2, NT, 8), jnp.float32),
        mesh=mesh,
        scratch_types=scratch,
        compiler_params=pltpu.CompilerParams(use_tc_tiling_on_sc=False),
    )


def _combine1_block(x_ref, a_ref, d_ref, s1s_ref, s1n_ref, s1b_ref,
                    s2n_ref, s2s_ref, s2b_ref, p_ref, self2_ref):
    agg = a_ref[0] + a_ref[1]
    deg = d_ref[0, :, 0:1] + d_ref[1, :, 0:1]
    rdeg = 1.0 / jnp.maximum(deg, 1.0)
    h = jnp.dot(x_ref[...], s1s_ref[...], preferred_element_type=jnp.float32)
    h = h + jnp.dot(agg * rdeg, s1n_ref[...],
                    preferred_element_type=jnp.float32)
    h = jnp.maximum(h + s1b_ref[...], 0.0)
    p_ref[...] = jnp.dot(h, s2n_ref[...], preferred_element_type=jnp.float32)
    self2_ref[...] = (jnp.dot(h, s2s_ref[...],
                              preferred_element_type=jnp.float32)
                      + s2b_ref[...])


def _combine1(x, aggp, degp, s1sT, s1nT, s1b, s2nT, s2sT, s2b):
    B = 1280
    return pl.pallas_call(
        _combine1_block,
        grid=(NP // B,),
        in_specs=[
            pl.BlockSpec((B, 32), lambda i: (i, 0)),
            pl.BlockSpec((2, B, 32), lambda i: (0, i, 0)),
            pl.BlockSpec((2, B, 8), lambda i: (0, i, 0)),
            pl.BlockSpec((32, 64), lambda i: (0, 0)),
            pl.BlockSpec((32, 64), lambda i: (0, 0)),
            pl.BlockSpec((1, 64), lambda i: (0, 0)),
            pl.BlockSpec((64, NCLS), lambda i: (0, 0)),
            pl.BlockSpec((64, NCLS), lambda i: (0, 0)),
            pl.BlockSpec((1, NCLS), lambda i: (0, 0)),
        ],
        out_specs=[
            pl.BlockSpec((B, NCLS), lambda i: (i, 0)),
            pl.BlockSpec((B, NCLS), lambda i: (i, 0)),
        ],
        out_shape=[
            jax.ShapeDtypeStruct((NP, NCLS), jnp.float32),
            jax.ShapeDtypeStruct((NP, NCLS), jnp.float32),
        ],
    )(x, aggp, degp, s1sT, s1nT, s1b, s2nT, s2sT, s2b)


def _combine2_block(self2_ref, a_ref, d_ref, out_ref):
    agg = a_ref[0] + a_ref[1]
    deg = d_ref[0, :, 0:1] + d_ref[1, :, 0:1]
    rdeg = 1.0 / jnp.maximum(deg, 1.0)
    out_ref[...] = self2_ref[...] + agg * rdeg


def _combine2(self2, aggp, degp):
    B = 1280
    return pl.pallas_call(
        _combine2_block,
        grid=(NP // B,),
        in_specs=[
            pl.BlockSpec((B, NCLS), lambda i: (i, 0)),
            pl.BlockSpec((2, B, NCLS), lambda i: (0, i, 0)),
            pl.BlockSpec((2, B, 8), lambda i: (0, i, 0)),
        ],
        out_specs=pl.BlockSpec((B, NCLS), lambda i: (i, 0)),
        out_shape=jax.ShapeDtypeStruct((NP, NCLS), jnp.float32),
    )(self2, aggp, degp)


def kernel(features, edge_index, conv_w, conv_b, lin1_w, lin1_b, lin2_w,
           lin2_b, s1_self, s1_neigh, s1_b, s2_self, s2_neigh, s2_b):
    f32 = jnp.float32

    wp = jnp.arange(64)[:, None]
    ww = jnp.arange(64)[None, :]
    off = wp - ww + 1
    valid = (off >= 0) & (off <= 2)
    offc = jnp.clip(off, 0, 2)
    bds = []
    for dh in range(3):
        tap = conv_w[:, 0, dh, :]
        M = tap[:, offc]
        M = jnp.where(valid[None], M, 0.0)
        bds.append(jnp.transpose(M, (1, 0, 2)).reshape(64, 2048))
    bcat = jnp.concatenate(bds, axis=0)
    brow = jnp.repeat(conv_b, 64).reshape(1, 2048)

    cols = jnp.arange(2048)
    obase = (cols // 64) * 320 + (cols % 64) // 2
    even = (cols % 64) % 2 == 0
    w1p = jnp.stack([
        jnp.where(even[:, None], lin1_w[:, obase + ph * 32].T, 0.0)
        for ph in range(10)
    ])
    b1r = lin1_b.reshape(1, 32)
    w2p = lin2_w.T
    b2r = lin2_b.reshape(1, 32)

    feats = jnp.pad(features, ((0, NP - N), (0, 0), (0, 0)))
    feats_t = feats.reshape(NP, 10, 2, 64).transpose(2, 1, 0, 3)

    x = _cnn(feats_t, bcat.astype(jnp.bfloat16), brow,
             w1p.astype(jnp.bfloat16), b1r, w2p, b2r)

    pad = EP - E
    srcp = jnp.concatenate([edge_index[0],
                            jnp.zeros((pad,), jnp.int32)]).reshape(NW, KCH, CH)
    dstp = jnp.concatenate([edge_index[1],
                            jnp.full((pad,), NP, jnp.int32)]).reshape(NW, KCH, CH)

    z32 = jnp.zeros((NT, 32), f32)
    z16 = jnp.zeros((NT, 16), f32)
    z8 = jnp.zeros((NT, 8), f32)
    ones8 = jnp.ones((CH, 8), f32)

    degp = _make_deg()(dstp, z8, ones8)
    agg1p = _make_scatter(32)(x, srcp, dstp, z32)

    p, self2 = _combine1(x, agg1p, degp, s1_self.T, s1_neigh.T,
                         s1_b.reshape(1, 64), s2_neigh.T, s2_self.T,
                         s2_b.reshape(1, NCLS))

    agg2p = _make_scatter(16)(p, srcp, dstp, z16)

    out = _combine2(self2, agg2p, degp)
    return out[:N]


# speedup vs baseline: 1.4170x; 1.0069x over previous
import functools

import jax
import jax.numpy as jnp
from jax import lax
from jax.experimental import pallas as pl
from jax.experimental.pallas import tpu as pltpu
from jax.experimental.pallas import tpu_sc as plsc

N = 10000
E = 640000
NCLS = 16

BN = 128
NP = 10240
NT = NP + 64
NW = 32
CH = 128
KCH = 160
EP = NW * KCH * CH


def _cnn_block(f_ref, bcat_ref, brow_ref, w1_ref, b1_ref, w2_ref,
               b2_ref, out_ref):
    Fe = f_ref[0]
    Fo = f_ref[1]
    rows = BN * 10
    zpl = jnp.zeros((1, BN, 64), jnp.float32)
    fo_dn = jnp.concatenate([zpl, Fo[:-1]], axis=0)
    fe_up = jnp.concatenate([Fe[1:], zpl], axis=0)
    gb = jnp.concatenate([fo_dn, Fe, Fo, fe_up],
                         axis=2).reshape(rows, 256).astype(jnp.bfloat16)
    ye = jnp.dot(gb[:, 0:192], bcat_ref[...],
                 preferred_element_type=jnp.float32)
    yo = jnp.dot(gb[:, 64:256], bcat_ref[...],
                 preferred_element_type=jnp.float32)
    yh = jnp.maximum(jnp.maximum(ye, yo) + brow_ref[...],
                     0.0).astype(jnp.bfloat16)
    yw = jnp.maximum(yh, pltpu.roll(yh, shift=2047, axis=1))
    y3 = yw.reshape(10, BN, 2048)
    acc = jnp.broadcast_to(b1_ref[...], (BN, 32))
    for k in range(10):
        acc = acc + jnp.dot(y3[k], w1_ref[k],
                            preferred_element_type=jnp.float32)
    z = jnp.maximum(acc, 0.0)
    x = jnp.dot(z, w2_ref[...], preferred_element_type=jnp.float32)
    out_ref[...] = jnp.maximum(x + b2_ref[...], 0.0)


def _cnn(feats_t, bcat, brow, w1p, b1r, w2p, b2r):
    return pl.pallas_call(
        _cnn_block,
        grid=(NP // BN,),
        in_specs=[
            pl.BlockSpec((2, 10, BN, 64), lambda i: (0, 0, i, 0)),
            pl.BlockSpec((192, 2048), lambda i: (0, 0)),
            pl.BlockSpec((1, 2048), lambda i: (0, 0)),
            pl.BlockSpec((10, 2048, 32), lambda i: (0, 0, 0)),
            pl.BlockSpec((1, 32), lambda i: (0, 0)),
            pl.BlockSpec((32, 32), lambda i: (0, 0)),
            pl.BlockSpec((1, 32), lambda i: (0, 0)),
        ],
        out_specs=pl.BlockSpec((BN, 32), lambda i: (i, 0)),
        out_shape=jax.ShapeDtypeStruct((NP, 32), jnp.float32),
    )(feats_t, bcat, brow, w1p, b1r, w2p, b2r)


def _make_scatter(D):
    mesh = plsc.VectorSubcoreMesh(core_axis_name="c", subcore_axis_name="s")
    out_type = jax.ShapeDtypeStruct((2, NT, D), jnp.float32)
    NB = 8
    HB = NB // 2
    scratch = [
        pltpu.VMEM((KCH, CH), jnp.int32),
        pltpu.VMEM((KCH, CH), jnp.int32),
    ]
    scratch += [pltpu.VMEM((CH, D), jnp.float32) for _ in range(NB)]
    scratch += [
        pltpu.VMEM_SHARED((NT, D), jnp.float32),
        pltpu.VMEM_SHARED((NP, D), jnp.float32),
    ]
    scratch += [pltpu.SemaphoreType.DMA for _ in range(2 * NB)]

    def body(*refs):
        (x_hbm, src_hbm, dst_hbm, z_hbm,
         agg_out, src_v, dst_v) = refs[:7]
        msg = list(refs[7:7 + NB])
        agg_sh = refs[7 + NB]
        x_sh = refs[8 + NB]
        gsem = list(refs[9 + NB:9 + 2 * NB])
        ssem = list(refs[9 + 2 * NB:9 + 3 * NB])
        c = lax.axis_index("c")
        s = lax.axis_index("s")
        wid = s * 2 + c
        pltpu.sync_copy(src_hbm.at[wid], src_v)
        pltpu.sync_copy(dst_hbm.at[wid], dst_v)
        rs = NP // 16
        pltpu.sync_copy(x_hbm.at[pl.ds(s * rs, rs)],
                        x_sh.at[pl.ds(s * rs, rs)])

        @pl.when(s == 0)
        def _init():
            pltpu.sync_copy(z_hbm, agg_sh)

        plsc.subcore_barrier()

        for b0 in range(HB):
            pltpu.async_copy(x_sh.at[src_v.at[b0]], msg[b0], gsem[b0])

        def do_chunk(t, b):
            b2 = (b + HB) % NB
            pltpu.make_async_copy(x_sh.at[src_v.at[t]], msg[b],
                                  gsem[b]).wait()
            pltpu.async_copy(msg[b], agg_sh.at[dst_v.at[t]], ssem[b],
                             add=True)

            @pl.when(t >= HB)
            def _wait_prev():
                pltpu.make_async_copy(msg[b2], agg_sh.at[dst_v.at[t]],
                                      ssem[b2]).wait()

            @pl.when(t + HB < KCH)
            def _next_gather():
                pltpu.async_copy(x_sh.at[src_v.at[t + HB]], msg[b2],
                                 gsem[b2])

        def step(tt, carry):
            for b in range(NB):
                do_chunk(tt * NB + b, b)
            return carry

        lax.fori_loop(0, KCH // NB, step, 0)
        for b in range(HB, NB):
            pltpu.make_async_copy(msg[b], agg_sh.at[dst_v.at[0]],
                                  ssem[b]).wait()
        plsc.subcore_barrier()

        @pl.when(s == 0)
        def _flush():
            pltpu.sync_copy(agg_sh, agg_out.at[c])

    return pl.kernel(
        body,
        out_type=out_type,
        mesh=mesh,
        scratch_types=scratch,
        compiler_params=pltpu.CompilerParams(use_tc_tiling_on_sc=False),
    )


def _make_deg():
    mesh = plsc.VectorSubcoreMesh(core_axis_name="c", subcore_axis_name="s")
    NB = 4
    scratch = [
        pltpu.VMEM((KCH, CH), jnp.int32),
        pltpu.VMEM((CH, 8), jnp.float32),
        pltpu.VMEM_SHARED((NT, 8), jnp.float32),
    ]
    scratch += [pltpu.SemaphoreType.DMA for _ in range(NB)]

    def body(*refs):
        (dst_hbm, z8_hbm, ones_hbm, deg_out, dst_v, ones_v, deg_sh) = refs[:7]
        dsem = list(refs[7:7 + NB])
        c = lax.axis_index("c")
        s = lax.axis_index("s")
        wid = s * 2 + c
        pltpu.sync_copy(dst_hbm.at[wid], dst_v)
        pltpu.sync_copy(ones_hbm, ones_v)

        @pl.when(s == 0)
        def _init():
            pltpu.sync_copy(z8_hbm, deg_sh)

        plsc.subcore_barrier()

        def do_chunk(t, b):
            @pl.when(t >= NB)
            def _wait_prev():
                pltpu.make_async_copy(ones_v, deg_sh.at[dst_v.at[t]],
                                      dsem[b]).wait()

            pltpu.async_copy(ones_v, deg_sh.at[dst_v.at[t]], dsem[b],
                             add=True)

        def step(tt, carry):
            for b in range(NB):
                do_chunk(tt * NB + b, b)
            return carry

        lax.fori_loop(0, KCH // NB, step, 0)
        for b in range(NB):
            pltpu.make_async_copy(ones_v, deg_sh.at[dst_v.at[0]],
                                  dsem[b]).wait()
        plsc.subcore_barrier()

        @pl.when(s == 0)
        def _flush():
            pltpu.sync_copy(deg_sh, deg_out.at[c])

    return pl.kernel(
        body,
        out_type=jax.ShapeDtypeStruct((2, NT, 8), jnp.float32),
        mesh=mesh,
        scratch_types=scratch,
        compiler_params=pltpu.CompilerParams(use_tc_tiling_on_sc=False),
    )


def _combine1_block(x_ref, a_ref, d_ref, s1s_ref, s1n_ref, s1b_ref,
                    s2n_ref, s2s_ref, s2b_ref, p_ref, self2_ref):
    agg = a_ref[0] + a_ref[1]
    deg = d_ref[0, :, 0:1] + d_ref[1, :, 0:1]
    rdeg = 1.0 / jnp.maximum(deg, 1.0)
    h = jnp.dot(x_ref[...], s1s_ref[...], preferred_element_type=jnp.float32)
    h = h + jnp.dot(agg * rdeg, s1n_ref[...],
                    preferred_element_type=jnp.float32)
    h = jnp.maximum(h + s1b_ref[...], 0.0)
    p_ref[...] = jnp.dot(h, s2n_ref[...], preferred_element_type=jnp.float32)
    self2_ref[...] = (jnp.dot(h, s2s_ref[...],
                              preferred_element_type=jnp.float32)
                      + s2b_ref[...])


def _combine1(x, aggp, degp, s1sT, s1nT, s1b, s2nT, s2sT, s2b):
    B = 1280
    return pl.pallas_call(
        _combine1_block,
        grid=(NP // B,),
        in_specs=[
            pl.BlockSpec((B, 32), lambda i: (i, 0)),
            pl.BlockSpec((2, B, 32), lambda i: (0, i, 0)),
            pl.BlockSpec((2, B, 8), lambda i: (0, i, 0)),
            pl.BlockSpec((32, 64), lambda i: (0, 0)),
            pl.BlockSpec((32, 64), lambda i: (0, 0)),
            pl.BlockSpec((1, 64), lambda i: (0, 0)),
            pl.BlockSpec((64, NCLS), lambda i: (0, 0)),
            pl.BlockSpec((64, NCLS), lambda i: (0, 0)),
            pl.BlockSpec((1, NCLS), lambda i: (0, 0)),
        ],
        out_specs=[
            pl.BlockSpec((B, NCLS), lambda i: (i, 0)),
            pl.BlockSpec((B, NCLS), lambda i: (i, 0)),
        ],
        out_shape=[
            jax.ShapeDtypeStruct((NP, NCLS), jnp.float32),
            jax.ShapeDtypeStruct((NP, NCLS), jnp.float32),
        ],
    )(x, aggp, degp, s1sT, s1nT, s1b, s2nT, s2sT, s2b)


def _combine2_block(self2_ref, a_ref, d_ref, out_ref):
    agg = a_ref[0] + a_ref[1]
    deg = d_ref[0, :, 0:1] + d_ref[1, :, 0:1]
    rdeg = 1.0 / jnp.maximum(deg, 1.0)
    out_ref[...] = self2_ref[...] + agg * rdeg


def _combine2(self2, aggp, degp):
    B = 1280
    return pl.pallas_call(
        _combine2_block,
        grid=(NP // B,),
        in_specs=[
            pl.BlockSpec((B, NCLS), lambda i: (i, 0)),
            pl.BlockSpec((2, B, NCLS), lambda i: (0, i, 0)),
            pl.BlockSpec((2, B, 8), lambda i: (0, i, 0)),
        ],
        out_specs=pl.BlockSpec((B, NCLS), lambda i: (i, 0)),
        out_shape=jax.ShapeDtypeStruct((NP, NCLS), jnp.float32),
    )(self2, aggp, degp)


def kernel(features, edge_index, conv_w, conv_b, lin1_w, lin1_b, lin2_w,
           lin2_b, s1_self, s1_neigh, s1_b, s2_self, s2_neigh, s2_b):
    f32 = jnp.float32

    wp = jnp.arange(64)[:, None]
    ww = jnp.arange(64)[None, :]
    off = wp - ww + 1
    valid = (off >= 0) & (off <= 2)
    offc = jnp.clip(off, 0, 2)
    bds = []
    for dh in range(3):
        tap = conv_w[:, 0, dh, :]
        M = tap[:, offc]
        M = jnp.where(valid[None], M, 0.0)
        bds.append(jnp.transpose(M, (1, 0, 2)).reshape(64, 2048))
    bcat = jnp.concatenate(bds, axis=0)
    brow = jnp.repeat(conv_b, 64).reshape(1, 2048)

    cols = jnp.arange(2048)
    obase = (cols // 64) * 320 + (cols % 64) // 2
    even = (cols % 64) % 2 == 0
    w1p = jnp.stack([
        jnp.where(even[:, None], lin1_w[:, obase + ph * 32].T, 0.0)
        for ph in range(10)
    ])
    b1r = lin1_b.reshape(1, 32)
    w2p = lin2_w.T
    b2r = lin2_b.reshape(1, 32)

    feats = jnp.pad(features, ((0, NP - N), (0, 0), (0, 0)))
    feats_t = feats.reshape(NP, 10, 2, 64).transpose(2, 1, 0, 3)

    x = _cnn(feats_t, bcat.astype(jnp.bfloat16), brow,
             w1p.astype(jnp.bfloat16), b1r, w2p, b2r)

    pad = EP - E
    srcp = jnp.concatenate([edge_index[0],
                            jnp.zeros((pad,), jnp.int32)]).reshape(NW, KCH, CH)
    dstp = jnp.concatenate([edge_index[1],
                            jnp.full((pad,), NP, jnp.int32)]).reshape(NW, KCH, CH)

    z32 = jnp.zeros((NT, 32), f32)
    z16 = jnp.zeros((NT, 16), f32)
    z8 = jnp.zeros((NT, 8), f32)
    ones8 = jnp.ones((CH, 8), f32)

    degp = _make_deg()(dstp, z8, ones8)
    agg1p = _make_scatter(32)(x, srcp, dstp, z32)

    p, self2 = _combine1(x, agg1p, degp, s1_self.T, s1_neigh.T,
                         s1_b.reshape(1, 64), s2_neigh.T, s2_self.T,
                         s2_b.reshape(1, NCLS))

    agg2p = _make_scatter(16)(p, srcp, dstp, z16)

    out = _combine2(self2, agg2p, degp)
    return out[:N]
